# baseline (speedup 1.0000x reference)
import jax
import jax.numpy as jnp
from jax.experimental import pallas as pl
from jax.experimental.pallas import tpu as pltpu

N = 16384
D = 64
TR = 512
TC = 512
NR = N // TR
NC = N // TC
INF = float("inf")


def _dot(x, y):
    return jax.lax.dot_general(
        x, y, (((1,), (1,)), ((), ())),
        preferred_element_type=jnp.float32,
        precision=jax.lax.Precision.DEFAULT,
    )


def _chamfer_kernel(
    cmap,
    rmap,
    pure,
    in_last,
    out_last,
    out_pts_ref,
    in_pts_ref,
    in_b_ref,
    out_b_ref,
    in_full_ref,
    out_full_ref,
    loss_ref,
    colmin_s,
    rowmin_s,
):
    i = pl.program_id(0)
    c = cmap[i]
    r = rmap[i]

    @pl.when(i == 0)
    def _init():
        colmin_s[:] = jnp.full((N,), INF, jnp.float32)
        rowmin_s[:] = jnp.full((N,), INF, jnp.float32)

    a = out_pts_ref[:]
    b = in_pts_ref[:]
    g1 = _dot(a, b * -2.0)
    g2 = _dot(b, a * -2.0)
    an1 = jnp.sum(a * a, axis=1)
    bn1 = jnp.sum(b * b, axis=1)
    t1 = an1[:, None] + g1
    t2 = bn1[:, None] + g2

    @pl.when(pure[i] == 1)
    def _pure_tile():
        cm = jnp.min(t1, axis=0) + bn1
        rm = jnp.min(t2, axis=0) + an1
        colmin_s[pl.ds(c * TC, TC)] = jnp.minimum(colmin_s[pl.ds(c * TC, TC)], cm)
        rowmin_s[pl.ds(r * TR, TR)] = jnp.minimum(rowmin_s[pl.ds(r * TR, TR)], rm)

    @pl.when(pure[i] == 0)
    def _mixed_tile():
        ob = out_b_ref[pl.ds(r * TR, TR)]
        ib = in_b_ref[pl.ds(c * TC, TC)]
        cm = jnp.min(jnp.where(ob[:, None] == ib[None, :], t1, INF), axis=0) + bn1
        rm = jnp.min(jnp.where(ib[:, None] == ob[None, :], t2, INF), axis=0) + an1
        colmin_s[pl.ds(c * TC, TC)] = jnp.minimum(colmin_s[pl.ds(c * TC, TC)], cm)
        rowmin_s[pl.ds(r * TR, TR)] = jnp.minimum(rowmin_s[pl.ds(r * TR, TR)], rm)

    @pl.when(i == pl.num_programs(0) - 1)
    def _final():
        in_b = in_b_ref[:]
        out_b = out_b_ref[:]
        nb = jnp.minimum(in_last[NC - 1], out_last[NR - 1])
        need = False
        for k in range(8):
            pin = jnp.any(in_b == k)
            pout = jnp.any(out_b == k)
            need = need | ((k < nb) & (pin != pout))

        @pl.when(jnp.logical_not(need))
        def _fast():
            loss = (jnp.sum(jnp.where(in_b < nb, colmin_s[:], 0.0))
                    + jnp.sum(jnp.where(out_b < nb, rowmin_s[:], 0.0)))
            loss_ref[:, :] = loss[None, None]

        @pl.when(need)
        def _fallback():
            din = in_full_ref[:] - out_full_ref[0, :][None, :]
            row0 = jnp.sum(din * din, axis=1)
            dout = out_full_ref[:] - in_full_ref[0, :][None, :]
            col0 = jnp.sum(dout * dout, axis=1)
            cmv = colmin_s[:]
            rmv = rowmin_s[:]
            cm_fixed = jnp.where(cmv < INF, cmv, row0)
            rm_fixed = jnp.where(rmv < INF, rmv, col0)
            loss = (jnp.sum(jnp.where(in_b < nb, cm_fixed, 0.0))
                    + jnp.sum(jnp.where(out_b < nb, rm_fixed, 0.0)))
            loss_ref[:, :] = loss[None, None]


def kernel(in_points_list, in_batch_list, out_points_list, out_batch_list):
    in_pts = in_points_list[0]
    out_pts = out_points_list[0]
    in_b = in_batch_list[0].astype(jnp.int32)
    out_b = out_batch_list[0].astype(jnp.int32)

    in_first = in_b[::TC]
    in_last = in_b[TC - 1::TC]
    out_first = out_b[::TR]
    out_last = out_b[TR - 1::TR]

    ov = (out_first[None, :] <= in_last[:, None]) & \
         (in_first[:, None] <= out_last[None, :])
    flat = ov.reshape(-1)
    n_active = jnp.sum(flat).astype(jnp.int32)
    pos = jnp.nonzero(flat, size=NC * NR, fill_value=0)[0].astype(jnp.int32)
    pos = jnp.concatenate([pos, jnp.zeros((1,), jnp.int32)])
    cmap = pos // NR
    rmap = pos % NR
    pure_flat = ((in_first == in_last)[:, None]
                 & (out_first == out_last)[None, :]
                 & (in_first[:, None] == out_first[None, :])).reshape(-1)
    pure = pure_flat.astype(jnp.int32)[pos]

    grid_spec = pltpu.PrefetchScalarGridSpec(
        num_scalar_prefetch=5,
        grid=(n_active + 1,),
        in_specs=[
            pl.BlockSpec((TR, D), lambda i, cm, rm, pu, il, ol: (rm[i], 0)),
            pl.BlockSpec((TC, D), lambda i, cm, rm, pu, il, ol: (cm[i], 0)),
            pl.BlockSpec((N,), lambda i, *_: (0,)),
            pl.BlockSpec((N,), lambda i, *_: (0,)),
            pl.BlockSpec((N, D), lambda i, *_: (0, 0)),
            pl.BlockSpec((N, D), lambda i, *_: (0, 0)),
        ],
        out_specs=pl.BlockSpec((1, 1), lambda i, *_: (0, 0)),
        scratch_shapes=[
            pltpu.VMEM((N,), jnp.float32),
            pltpu.VMEM((N,), jnp.float32),
        ],
    )
    loss = pl.pallas_call(
        _chamfer_kernel,
        grid_spec=grid_spec,
        out_shape=jax.ShapeDtypeStruct((1, 1), jnp.float32),
        compiler_params=pltpu.CompilerParams(
            dimension_semantics=("arbitrary",),
        ),
    )(cmap, rmap, pure, in_last, out_last,
      out_pts, in_pts, in_b, out_b, in_pts, out_pts)
    return loss[0, 0]


# speedup vs baseline: 1.0038x; 1.0038x over previous
import jax
import jax.numpy as jnp
from jax.experimental import pallas as pl
from jax.experimental.pallas import tpu as pltpu

N = 16384
D = 64
TR = 512
TC = 512
NR = N // TR
NC = N // TC
INF = float("inf")


def _dot(x, y):
    return jax.lax.dot_general(
        x, y, (((1,), (1,)), ((), ())),
        preferred_element_type=jnp.float32,
        precision=jax.lax.Precision.DEFAULT,
    )


def _chamfer_kernel(
    cmap,
    rmap,
    pure,
    in_last,
    out_last,
    out_pts_ref,
    in_pts_ref,
    in_b_ref,
    out_b_ref,
    in_full_ref,
    out_full_ref,
    loss_ref,
    colmin_s,
    rowmin_s,
):
    i = pl.program_id(0)
    c = cmap[i]
    r = rmap[i]

    @pl.when(i == 0)
    def _init():
        colmin_s[:] = jnp.full((N,), INF, jnp.float32)
        rowmin_s[:] = jnp.full((N,), INF, jnp.float32)

    a = out_pts_ref[:]
    b = in_pts_ref[:]
    g1 = _dot(a, b * -2.0)
    g2 = _dot(b, a * -2.0)
    an1 = jnp.sum(a * a, axis=1)
    bn1 = jnp.sum(b * b, axis=1)
    t1 = (an1[:, None] + g1) + bn1[None, :]
    t2 = (bn1[:, None] + g2) + an1[None, :]

    @pl.when(pure[i] == 1)
    def _pure_tile():
        cm = jnp.min(t1, axis=0)
        rm = jnp.min(t2, axis=0)
        colmin_s[pl.ds(c * TC, TC)] = jnp.minimum(colmin_s[pl.ds(c * TC, TC)], cm)
        rowmin_s[pl.ds(r * TR, TR)] = jnp.minimum(rowmin_s[pl.ds(r * TR, TR)], rm)

    @pl.when(pure[i] == 0)
    def _mixed_tile():
        ob = out_b_ref[pl.ds(r * TR, TR)]
        ib = in_b_ref[pl.ds(c * TC, TC)]
        cm = jnp.min(jnp.where(ob[:, None] == ib[None, :], t1, INF), axis=0)
        rm = jnp.min(jnp.where(ib[:, None] == ob[None, :], t2, INF), axis=0)
        colmin_s[pl.ds(c * TC, TC)] = jnp.minimum(colmin_s[pl.ds(c * TC, TC)], cm)
        rowmin_s[pl.ds(r * TR, TR)] = jnp.minimum(rowmin_s[pl.ds(r * TR, TR)], rm)

    @pl.when(i == pl.num_programs(0) - 1)
    def _final():
        in_b = in_b_ref[:]
        out_b = out_b_ref[:]
        nb = jnp.minimum(in_last[NC - 1], out_last[NR - 1])
        need = False
        for k in range(8):
            pin = jnp.any(in_b == k)
            pout = jnp.any(out_b == k)
            need = need | ((k < nb) & (pin != pout))

        @pl.when(jnp.logical_not(need))
        def _fast():
            loss = (jnp.sum(jnp.where(in_b < nb, colmin_s[:], 0.0))
                    + jnp.sum(jnp.where(out_b < nb, rowmin_s[:], 0.0)))
            loss_ref[:, :] = loss[None, None]

        @pl.when(need)
        def _fallback():
            din = in_full_ref[:] - out_full_ref[0, :][None, :]
            row0 = jnp.sum(din * din, axis=1)
            dout = out_full_ref[:] - in_full_ref[0, :][None, :]
            col0 = jnp.sum(dout * dout, axis=1)
            cmv = colmin_s[:]
            rmv = rowmin_s[:]
            cm_fixed = jnp.where(cmv < INF, cmv, row0)
            rm_fixed = jnp.where(rmv < INF, rmv, col0)
            loss = (jnp.sum(jnp.where(in_b < nb, cm_fixed, 0.0))
                    + jnp.sum(jnp.where(out_b < nb, rm_fixed, 0.0)))
            loss_ref[:, :] = loss[None, None]


def kernel(in_points_list, in_batch_list, out_points_list, out_batch_list):
    in_pts = in_points_list[0]
    out_pts = out_points_list[0]
    in_b = in_batch_list[0].astype(jnp.int32)
    out_b = out_batch_list[0].astype(jnp.int32)

    in_first = in_b[::TC]
    in_last = in_b[TC - 1::TC]
    out_first = out_b[::TR]
    out_last = out_b[TR - 1::TR]

    ov = (out_first[None, :] <= in_last[:, None]) & \
         (in_first[:, None] <= out_last[None, :])
    flat = ov.reshape(-1)
    n_active = jnp.sum(flat).astype(jnp.int32)
    pos = jnp.nonzero(flat, size=NC * NR, fill_value=0)[0].astype(jnp.int32)
    pos = jnp.concatenate([pos, jnp.zeros((1,), jnp.int32)])
    cmap = pos // NR
    rmap = pos % NR
    pure_flat = ((in_first == in_last)[:, None]
                 & (out_first == out_last)[None, :]
                 & (in_first[:, None] == out_first[None, :])).reshape(-1)
    pure = pure_flat.astype(jnp.int32)[pos]

    grid_spec = pltpu.PrefetchScalarGridSpec(
        num_scalar_prefetch=5,
        grid=(n_active + 1,),
        in_specs=[
            pl.BlockSpec((TR, D), lambda i, cm, rm, pu, il, ol: (rm[i], 0)),
            pl.BlockSpec((TC, D), lambda i, cm, rm, pu, il, ol: (cm[i], 0)),
            pl.BlockSpec((N,), lambda i, *_: (0,)),
            pl.BlockSpec((N,), lambda i, *_: (0,)),
            pl.BlockSpec((N, D), lambda i, *_: (0, 0)),
            pl.BlockSpec((N, D), lambda i, *_: (0, 0)),
        ],
        out_specs=pl.BlockSpec((1, 1), lambda i, *_: (0, 0)),
        scratch_shapes=[
            pltpu.VMEM((N,), jnp.float32),
            pltpu.VMEM((N,), jnp.float32),
        ],
    )
    loss = pl.pallas_call(
        _chamfer_kernel,
        grid_spec=grid_spec,
        out_shape=jax.ShapeDtypeStruct((1, 1), jnp.float32),
        compiler_params=pltpu.CompilerParams(
            dimension_semantics=("arbitrary",),
        ),
    )(cmap, rmap, pure, in_last, out_last,
      out_pts, in_pts, in_b, out_b, in_pts, out_pts)
    return loss[0, 0]
